# Initial kernel scaffold; baseline (speedup 1.0000x reference)
import jax
import jax.numpy as jnp
from jax.experimental import pallas as pl


def kernel(x, mask):
    raise NotImplementedError("write your pallas kernel here")


# speedup vs baseline: 6.6842x; 6.6842x over previous
import functools

import jax
import jax.numpy as jnp
from jax import lax
from jax.experimental import pallas as pl
from jax.experimental.pallas import tpu as pltpu, tpu_sc as plsc

B_EFF = 128
N_PATCH = 256
P_ELEMS = 256
M = 128

NC = 2
NS = 16
NW = NC * NS
B_PER_W = B_EFF // NW
L = 16


def _seft_body(x_hbm, mask_hbm, out_hbm, mask_v, idx_v, rowid_v, buf_v, sem):
    wid = lax.axis_index("s") * NC + lax.axis_index("c")

    pltpu.sync_copy(mask_hbm, mask_v)
    for i in range(M // L):
        m = mask_v[pl.ds(i * L, L)]
        idx_v[pl.ds(i * L, L)] = plsc.load_gather(mask_v, [m])

    def start_gather(k, slot):
        b = wid * B_PER_W + k
        for i in range(M // L):
            rowid_v[slot, pl.ds(i * L, L)] = idx_v[pl.ds(i * L, L)] + b * N_PATCH
        return pltpu.async_copy(x_hbm.at[rowid_v.at[slot]], buf_v.at[slot], sem)

    cp = start_gather(0, 0)
    for k in range(B_PER_W):
        slot = k % 2
        cp.wait()
        if k + 1 < B_PER_W:
            cp = start_gather(k + 1, (k + 1) % 2)
        b = wid * B_PER_W + k
        pltpu.sync_copy(buf_v.at[slot], out_hbm.at[pl.ds(b * M, M)])


@functools.partial(jax.jit, static_argnames=())
def kernel(x, mask):
    x_flat = x.reshape(B_EFF * N_PATCH, P_ELEMS)
    mesh = plsc.VectorSubcoreMesh(core_axis_name="c", subcore_axis_name="s")
    k = functools.partial(
        pl.kernel,
        mesh=mesh,
        compiler_params=pltpu.CompilerParams(needs_layout_passes=False),
        out_type=jax.ShapeDtypeStruct((B_EFF * M, P_ELEMS), jnp.float32),
        scratch_types=[
            pltpu.VMEM((M,), jnp.int32),
            pltpu.VMEM((M,), jnp.int32),
            pltpu.VMEM((2, M), jnp.int32),
            pltpu.VMEM((2, M, P_ELEMS), jnp.float32),
            pltpu.SemaphoreType.DMA,
        ],
    )(_seft_body)
    out = k(x_flat, mask)
    return out.reshape(B_EFF, M, 4, 16, 4)


# speedup vs baseline: 6.6871x; 1.0004x over previous
import functools

import jax
import jax.numpy as jnp
from jax import lax
from jax.experimental import pallas as pl
from jax.experimental.pallas import tpu as pltpu, tpu_sc as plsc

B_EFF = 128
N_PATCH = 256
P_ELEMS = 256
M = 128

NC = 2
NS = 16
NW = NC * NS
B_PER_W = B_EFF // NW
L = 16


def _seft_body(x_hbm, mask_hbm, out_hbm, mask_v, rowid0, rowid1, buf_v, sem,
               wsem):
    wid = lax.axis_index("s") * NC + lax.axis_index("c")
    rowids = (rowid0, rowid1)

    pltpu.sync_copy(mask_hbm, mask_v)

    def fill_rowids(k, slot):
        b = wid * B_PER_W + k
        for i in range(M // L):
            m = mask_v[pl.ds(i * L, L)]
            idx = plsc.load_gather(mask_v, [m])
            rowids[slot][pl.ds(i * L, L)] = idx + b * N_PATCH

    def start_gather(k, slot):
        fill_rowids(k, slot)
        return pltpu.async_copy(x_hbm.at[rowids[slot]], buf_v.at[slot], sem)

    cp = start_gather(0, 0)
    wcp = None
    for k in range(B_PER_W):
        slot = k % 2
        cp.wait()
        if k + 1 < B_PER_W:
            cp = start_gather(k + 1, (k + 1) % 2)
        b = wid * B_PER_W + k
        if wcp is not None:
            wcp.wait()
        wcp = pltpu.async_copy(
            buf_v.at[slot], out_hbm.at[pl.ds(b * M, M)], wsem)
    wcp.wait()


@functools.partial(jax.jit, static_argnames=())
def kernel(x, mask):
    x_flat = x.reshape(B_EFF * N_PATCH, P_ELEMS)
    mesh = plsc.VectorSubcoreMesh(core_axis_name="c", subcore_axis_name="s")
    k = functools.partial(
        pl.kernel,
        mesh=mesh,
        compiler_params=pltpu.CompilerParams(needs_layout_passes=False),
        out_type=jax.ShapeDtypeStruct((B_EFF * M, P_ELEMS), jnp.float32),
        scratch_types=[
            pltpu.VMEM((M,), jnp.int32),
            pltpu.VMEM((M,), jnp.int32),
            pltpu.VMEM((M,), jnp.int32),
            pltpu.VMEM((2, M, P_ELEMS), jnp.float32),
            pltpu.SemaphoreType.DMA,
            pltpu.SemaphoreType.DMA,
        ],
    )(_seft_body)
    out = k(x_flat, mask)
    return out.reshape(B_EFF, M, 4, 16, 4)


# speedup vs baseline: 11.6788x; 1.7465x over previous
import functools

import jax
import jax.numpy as jnp
from jax import lax
from jax.experimental import pallas as pl
from jax.experimental.pallas import tpu as pltpu, tpu_sc as plsc

NC = 2
NS = 16
L = 16
M = 128


def _seft_body(x_hbm, mask_hbm, out_hbm,
               mask_v, j_v, mrow_v, rowids, m_buf, out_buf, sem, wsem):
    wid = lax.axis_index("s") * NC + lax.axis_index("c")

    pltpu.sync_copy(mask_hbm, mask_v)
    for c in range(M // L):
        mv = mask_v[pl.ds(c * L, L)]
        idx = plsc.load_gather(mask_v, [mv])
        j_v[pl.ds(c * L, L)] = lax.shift_right_logical(idx, 4)
        mrow_v[pl.ds(c * L, L)] = lax.shift_left(idx & 15, 5)

    j_k = [j_v[pl.ds(k * L, L)] for k in range(8)]
    mrow_k = [mrow_v[pl.ds(k * L, L)] for k in range(8)]

    iota = lax.iota(jnp.int32, L)
    pat0 = lax.shift_left(lax.shift_right_logical(iota, 3), 7) + (iota & 7)
    pat1 = pat0 + 256

    wcps = []
    for unit in range(2):
        u = wid * 2 + unit
        grp = u // 4
        hp = u % 4
        for half in range(2):
            grp_hp = hp * 1024 + grp * 8 + half * 512
            for m in range(16):
                base = ((m // 4) * 16 + (m % 4) * 4) * 1024 + grp_hp
                rowids[m, pl.ds(0, L)] = pat0 + base
                rowids[m, pl.ds(L, L)] = pat1 + base
            cps = [
                pltpu.async_copy(
                    x_hbm.at[rowids.at[m]], m_buf.at[pl.ds(m * 32, 32)], sem)
                for m in range(16)
            ]
            for wcp in wcps:
                wcp.wait()
            wcps = []
            for cp in cps:
                cp.wait()

            def compute(wp, carry):
                rv = [mrow_k[k] + wp for k in range(8)]
                for b in range(8):
                    row = b * 32 + wp
                    for k in range(8):
                        g = plsc.load_gather(m_buf, [rv[k], j_k[k] + b * L])
                        out_buf[row, pl.ds(k * L, L)] = g
                return carry

            lax.fori_loop(0, 32, compute, 0)

            obase = grp * 2048 + hp * 64 + half * 32
            wcps = [
                pltpu.async_copy(
                    out_buf.at[pl.ds(b * 32, 32)],
                    out_hbm.at[pl.ds(obase + b * 256, 32)], wsem)
                for b in range(8)
            ]
    for wcp in wcps:
        wcp.wait()


@jax.jit
def kernel(x, mask):
    xe = x.transpose(1, 2, 3, 0)
    xe = xe.reshape(4, 16, 8, 8, 16, 128)
    xe = xe.transpose(0, 1, 2, 4, 3, 5)
    xe = xe.reshape(65536, 128)

    mesh = plsc.VectorSubcoreMesh(core_axis_name="c", subcore_axis_name="s")
    k = functools.partial(
        pl.kernel,
        mesh=mesh,
        compiler_params=pltpu.CompilerParams(needs_layout_passes=False),
        out_type=jax.ShapeDtypeStruct((32768, 128), jnp.float32),
        scratch_types=[
            pltpu.VMEM((M,), jnp.int32),
            pltpu.VMEM((M,), jnp.int32),
            pltpu.VMEM((M,), jnp.int32),
            pltpu.VMEM((16, 32), jnp.int32),
            pltpu.VMEM((512, 128), jnp.float32),
            pltpu.VMEM((256, 128), jnp.float32),
            pltpu.SemaphoreType.DMA,
            pltpu.SemaphoreType.DMA,
        ],
    )(_seft_body)
    out = k(xe, mask)
    return out.reshape(128, 4, 16, 4, 128).transpose(0, 4, 1, 2, 3)


# speedup vs baseline: 13.0442x; 1.1169x over previous
import functools

import jax
import jax.numpy as jnp
from jax import lax
from jax.experimental import pallas as pl
from jax.experimental.pallas import tpu as pltpu, tpu_sc as plsc

NC = 2
NS = 16
L = 16
M = 128
NPH = 8


def _seft_body(x_hbm, mask_hbm, out_hbm, mask_v, j_v, mrow_v, rowids,
               m_buf, out_buf, sem0, sem1, wsem0, wsem1):
    wid = lax.axis_index("s") * NC + lax.axis_index("c")
    sems = (sem0, sem1)
    wsems = (wsem0, wsem1)

    pltpu.sync_copy(mask_hbm, mask_v)
    for c in range(M // L):
        mv = mask_v[pl.ds(c * L, L)]
        idx = plsc.load_gather(mask_v, [mv])
        j_v[pl.ds(c * L, L)] = lax.shift_right_logical(idx, 4)
        mrow_v[pl.ds(c * L, L)] = lax.shift_left(idx & 15, 4)

    j_k = [j_v[pl.ds(k * L, L)] for k in range(8)]
    mrow_k = [mrow_v[pl.ds(k * L, L)] for k in range(8)]

    iota = lax.iota(jnp.int32, L)
    pat = lax.shift_left(lax.shift_right_logical(iota, 3), 7) + (iota & 7)

    def fire_stage(ph):
        pp = ph % 2
        u = wid * 2 + ph // 4
        grp = u // 4
        hp = u % 4
        base_u = hp * 1024 + grp * 8 + (ph % 4) * 256 + pat
        for m in range(16):
            s = m // 8
            rowids[pp * 2 + s, pl.ds((m % 8) * L, L)] = (
                base_u + ((m // 4) * 16 + (m % 4) * 4) * 1024)
        return [
            pltpu.async_copy(
                x_hbm.at[rowids.at[pp * 2 + s]],
                m_buf.at[pl.ds(pp * 256 + s * 128, 128)], sems[pp])
            for s in range(2)
        ]

    stage_cps = fire_stage(0)
    write_cps = {}
    for ph in range(NPH):
        pp = ph % 2
        next_cps = fire_stage(ph + 1) if ph + 1 < NPH else []
        for cp in write_cps.pop(pp, []):
            cp.wait()
        for cp in stage_cps:
            cp.wait()
        stage_cps = next_cps

        mofs = pp * 256
        oofs = pp * 128

        def compute(wp, carry):
            rv = [mrow_k[k] + (mofs + wp) for k in range(8)]
            for b in range(8):
                row = oofs + b * L + wp
                for k in range(8):
                    g = plsc.load_gather(m_buf, [rv[k], j_k[k] + b * L])
                    out_buf[row, pl.ds(k * L, L)] = g
            return carry

        lax.fori_loop(0, L, compute, 0)

        u = wid * 2 + ph // 4
        obase = (u // 4) * 2048 + (u % 4) * 64 + (ph % 4) * L
        write_cps[pp] = [
            pltpu.async_copy(
                out_buf.at[pl.ds(oofs + b * L, L)],
                out_hbm.at[pl.ds(obase + b * 256, L)], wsems[pp])
            for b in range(8)
        ]
    for cps in write_cps.values():
        for cp in cps:
            cp.wait()


@jax.jit
def kernel(x, mask):
    xe = x.transpose(1, 2, 3, 0)
    xe = xe.reshape(4, 16, 8, 8, 16, 128)
    xe = xe.transpose(0, 1, 2, 4, 3, 5)
    xe = xe.reshape(65536, 128)

    mesh = plsc.VectorSubcoreMesh(core_axis_name="c", subcore_axis_name="s")
    k = functools.partial(
        pl.kernel,
        mesh=mesh,
        compiler_params=pltpu.CompilerParams(needs_layout_passes=False),
        out_type=jax.ShapeDtypeStruct((32768, 128), jnp.float32),
        scratch_types=[
            pltpu.VMEM((M,), jnp.int32),
            pltpu.VMEM((M,), jnp.int32),
            pltpu.VMEM((M,), jnp.int32),
            pltpu.VMEM((4, M), jnp.int32),
            pltpu.VMEM((512, 128), jnp.float32),
            pltpu.VMEM((256, 128), jnp.float32),
            pltpu.SemaphoreType.DMA,
            pltpu.SemaphoreType.DMA,
            pltpu.SemaphoreType.DMA,
            pltpu.SemaphoreType.DMA,
        ],
    )(_seft_body)
    out = k(xe, mask)
    return out.reshape(128, 4, 16, 4, 128).transpose(0, 4, 1, 2, 3)


# speedup vs baseline: 22.8516x; 1.7519x over previous
import functools

import jax
import jax.numpy as jnp
from jax import lax
from jax.experimental import pallas as pl
from jax.experimental.pallas import tpu as pltpu, tpu_sc as plsc

NC = 2
NS = 16
L = 16
M = 128
NPH = 8


def _seft_body(x_hbm, mask_hbm, out_hbm, mask_v, j_v, mrow_v, rowids,
               m_buf, out_buf, sem0, sem1, wsem0, wsem1):
    wid = lax.axis_index("s") * NC + lax.axis_index("c")
    sems = (sem0, sem1)
    wsems = (wsem0, wsem1)

    pltpu.sync_copy(mask_hbm, mask_v)
    for c in range(M // L):
        mv = mask_v[pl.ds(c * L, L)]
        idx = plsc.load_gather(mask_v, [mv])
        j_v[pl.ds(c * L, L)] = lax.shift_right_logical(idx, 4)
        mrow_v[pl.ds(c * L, L)] = lax.shift_left(idx & 15, 4)

    j_k = [j_v[pl.ds(k * L, L)] for k in range(8)]
    mrow_k = [mrow_v[pl.ds(k * L, L)] for k in range(8)]

    iota = lax.iota(jnp.int32, L)
    pat = lax.shift_left(lax.shift_right_logical(iota, 3), 7) + (iota & 7)

    def fire_stage(ph):
        pp = ph % 2
        u = wid * 2 + ph // 4
        grp = u // 4
        hp = u % 4
        base_u = hp * 1024 + grp * 8 + (ph % 4) * 256 + pat
        for m in range(16):
            s = m // 8
            rowids[pp * 2 + s, pl.ds((m % 8) * L, L)] = (
                base_u + ((m // 4) * 16 + (m % 4) * 4) * 1024)
        return [
            pltpu.async_copy(
                x_hbm.at[rowids.at[pp * 2 + s]],
                m_buf.at[pl.ds(pp * 256 + s * 128, 128)], sems[pp])
            for s in range(2)
        ]

    stage_cps = fire_stage(0)
    write_cps = {}
    for ph in range(NPH):
        pp = ph % 2
        next_cps = fire_stage(ph + 1) if ph + 1 < NPH else []
        for cp in write_cps.pop(pp, []):
            cp.wait()
        for cp in stage_cps:
            cp.wait()
        stage_cps = next_cps

        mofs = pp * 256
        oofs = pp * 128

        def compute(wp, carry):
            rv = [mrow_k[k] + (mofs + wp) for k in range(8)]
            for b in range(8):
                row = oofs + b * L + wp
                gs = [plsc.load_gather(m_buf, [rv[k], j_k[k] + b * L])
                      for k in range(8)]
                for k in range(8):
                    out_buf[row, pl.ds(k * L, L)] = gs[k]
            return carry

        lax.fori_loop(0, L, compute, 0)

        u = wid * 2 + ph // 4
        obase = (u // 4) * 2048 + (u % 4) * 64 + (ph % 4) * L
        write_cps[pp] = [
            pltpu.async_copy(
                out_buf.at[pl.ds(oofs + b * L, L)],
                out_hbm.at[pl.ds(obase + b * 256, L)], wsems[pp])
            for b in range(8)
        ]
    for cps in write_cps.values():
        for cp in cps:
            cp.wait()


@jax.jit
def kernel(x, mask):
    xe = x.transpose(1, 2, 3, 0)
    xe = xe.reshape(4, 16, 8, 8, 16, 128)
    xe = xe.transpose(0, 1, 2, 4, 3, 5)
    xe = xe.reshape(65536, 128)

    mesh = plsc.VectorSubcoreMesh(core_axis_name="c", subcore_axis_name="s")
    k = functools.partial(
        pl.kernel,
        mesh=mesh,
        compiler_params=pltpu.CompilerParams(needs_layout_passes=False),
        out_type=jax.ShapeDtypeStruct((32768, 128), jnp.float32),
        scratch_types=[
            pltpu.VMEM((M,), jnp.int32),
            pltpu.VMEM((M,), jnp.int32),
            pltpu.VMEM((M,), jnp.int32),
            pltpu.VMEM((4, M), jnp.int32),
            pltpu.VMEM((512, 128), jnp.float32),
            pltpu.VMEM((256, 128), jnp.float32),
            pltpu.SemaphoreType.DMA,
            pltpu.SemaphoreType.DMA,
            pltpu.SemaphoreType.DMA,
            pltpu.SemaphoreType.DMA,
        ],
    )(_seft_body)
    out = k(xe, mask)
    return out.reshape(128, 4, 16, 4, 128).transpose(0, 4, 1, 2, 3)


# speedup vs baseline: 23.2263x; 1.0164x over previous
import functools

import jax
import jax.numpy as jnp
from jax import lax
from jax.experimental import pallas as pl
from jax.experimental.pallas import tpu as pltpu, tpu_sc as plsc

NC = 2
NS = 16
L = 16
M = 128
NPH = 8


def _seft_body(x_hbm, mask_hbm, out_hbm, mask_v, j_v, mrow_v, rowids,
               m_buf, out_buf, sem0, sem1, wsem0, wsem1):
    wid = lax.axis_index("s") * NC + lax.axis_index("c")
    sems = (sem0, sem1)
    wsems = (wsem0, wsem1)

    iota = lax.iota(jnp.int32, L)
    pat = lax.shift_left(lax.shift_right_logical(iota, 3), 7) + (iota & 7)

    def fire_stage(ph):
        pp = ph % 2
        u = wid * 2 + ph // 4
        grp = u // 4
        hp = u % 4
        base_u = hp * 1024 + grp * 8 + (ph % 4) * 256 + pat
        for m in range(16):
            s = m // 8
            rowids[pp * 2 + s, pl.ds((m % 8) * L, L)] = (
                base_u + ((m // 4) * 16 + (m % 4) * 4) * 1024)
        return [
            pltpu.async_copy(
                x_hbm.at[rowids.at[pp * 2 + s]],
                m_buf.at[pl.ds(pp * 256 + s * 128, 128)], sems[pp])
            for s in range(2)
        ]

    stage_cps = {0: fire_stage(0), 1: fire_stage(1)}

    pltpu.sync_copy(mask_hbm, mask_v)
    for c in range(M // L):
        mv = mask_v[pl.ds(c * L, L)]
        idx = plsc.load_gather(mask_v, [mv])
        j_v[pl.ds(c * L, L)] = lax.shift_right_logical(idx, 4)
        mrow_v[pl.ds(c * L, L)] = lax.shift_left(idx & 15, 4)

    j_k = [j_v[pl.ds(k * L, L)] for k in range(8)]
    mrow_k = [mrow_v[pl.ds(k * L, L)] for k in range(8)]

    write_cps = {}
    for ph in range(NPH):
        pp = ph % 2
        for cp in write_cps.pop(pp, []):
            cp.wait()
        for cp in stage_cps.pop(ph):
            cp.wait()

        mofs = pp * 256
        oofs = pp * 128

        def compute(wp, carry):
            rv = [mrow_k[k] + (mofs + wp) for k in range(8)]
            for b in range(8):
                row = oofs + b * L + wp
                gs = [plsc.load_gather(m_buf, [rv[k], j_k[k] + b * L])
                      for k in range(8)]
                for k in range(8):
                    out_buf[row, pl.ds(k * L, L)] = gs[k]
            return carry

        lax.fori_loop(0, L, compute, 0)

        if ph + 2 < NPH:
            stage_cps[ph + 2] = fire_stage(ph + 2)

        u = wid * 2 + ph // 4
        obase = (u // 4) * 2048 + (u % 4) * 64 + (ph % 4) * L
        write_cps[pp] = [
            pltpu.async_copy(
                out_buf.at[pl.ds(oofs + b * L, L)],
                out_hbm.at[pl.ds(obase + b * 256, L)], wsems[pp])
            for b in range(8)
        ]
    for cps in write_cps.values():
        for cp in cps:
            cp.wait()


@jax.jit
def kernel(x, mask):
    xe = x.transpose(1, 2, 3, 0)
    xe = xe.reshape(4, 16, 8, 8, 16, 128)
    xe = xe.transpose(0, 1, 2, 4, 3, 5)
    xe = xe.reshape(65536, 128)

    mesh = plsc.VectorSubcoreMesh(core_axis_name="c", subcore_axis_name="s")
    k = functools.partial(
        pl.kernel,
        mesh=mesh,
        compiler_params=pltpu.CompilerParams(needs_layout_passes=False),
        out_type=jax.ShapeDtypeStruct((32768, 128), jnp.float32),
        scratch_types=[
            pltpu.VMEM((M,), jnp.int32),
            pltpu.VMEM((M,), jnp.int32),
            pltpu.VMEM((M,), jnp.int32),
            pltpu.VMEM((4, M), jnp.int32),
            pltpu.VMEM((512, 128), jnp.float32),
            pltpu.VMEM((256, 128), jnp.float32),
            pltpu.SemaphoreType.DMA,
            pltpu.SemaphoreType.DMA,
            pltpu.SemaphoreType.DMA,
            pltpu.SemaphoreType.DMA,
        ],
    )(_seft_body)
    out = k(xe, mask)
    return out.reshape(128, 4, 16, 4, 128).transpose(0, 4, 1, 2, 3)
